# Initial kernel scaffold; baseline (speedup 1.0000x reference)
import jax
import jax.numpy as jnp
from jax.experimental import pallas as pl


def kernel(x, edge_index):
    raise NotImplementedError("write your pallas kernel here")


# speedup vs baseline: 11.7239x; 11.7239x over previous
import functools

import jax
import jax.numpy as jnp
from jax import lax
from jax.experimental import pallas as pl
from jax.experimental.pallas import tpu as pltpu
from jax.experimental.pallas import tpu_sc as plsc

N = 10000
D = 256
DH = 128
E = 160000
NS = 16
NC = 2
NPAD = 10240
EBLK = 64
EPAD = 163840
NBLK = EPAD // EBLK
RT = NPAD // NS
BT = NBLK // NS
NLAST = N - (NS - 1) * RT
WCH = 40
RSQRT_MAGIC = 0x5F3759DF


def _scale_rows_by_dis(buf, disb, dis_off, nrows):
    def _row(i, _):
        d = disb[dis_off + i, :][0]
        for k in range(DH // 16):
            sl = pl.ds(16 * k, 16)
            buf[i, sl] = buf[i, sl] * d
        return 0

    lax.fori_loop(0, nrows, _row, 0)


@functools.cache
def _build_kernels():
    mesh = plsc.VectorSubcoreMesh(
        core_axis_name="c", subcore_axis_name="s", num_cores=NC, num_subcores=NS
    )

    @functools.partial(
        pl.kernel,
        mesh=mesh,
        compiler_params=pltpu.CompilerParams(use_tc_tiling_on_sc=False),
        out_type=[
            jax.ShapeDtypeStruct((NC, NPAD, DH), jnp.float32),
            jax.ShapeDtypeStruct((NC, NPAD, 16), jnp.float32),
        ],
        scratch_types=[
            pltpu.VMEM((BT, EBLK), jnp.int32),
            pltpu.VMEM((EBLK, 16), jnp.float32),
            pltpu.VMEM((RT, 16), jnp.float32),
            pltpu.VMEM((RT, DH), jnp.float32),
            pltpu.VMEM_SHARED((NPAD, 16), jnp.float32),
        ],
    )
    def deg_prescale(x3, row2, zeros16, y2, dis2, ridx, ones, disb, xbuf, deg_acc):
        c = lax.axis_index("c")
        s = lax.axis_index("s")
        rbase = s * RT
        ebase = s * BT

        pltpu.sync_copy(zeros16, deg_acc.at[pl.ds(rbase, RT)])

        pltpu.sync_copy(row2.at[pl.ds(ebase, BT)], ridx)
        onev = jnp.full((16,), 1.0, jnp.float32)

        def _fill(j, _):
            ones[j, :] = onev
            return 0

        lax.fori_loop(0, EBLK, _fill, 0)
        plsc.subcore_barrier()

        def _hist(j, _):
            pltpu.sync_copy(ones, deg_acc.at[ridx.at[j]], add=True)
            return 0

        lax.fori_loop(0, BT, _hist, 0)
        plsc.subcore_barrier()

        pltpu.sync_copy(deg_acc.at[pl.ds(rbase, RT)], disb)

        def _rsqrt(i, _):
            v = disb[i, :]
            u = lax.bitcast_convert_type(v, jnp.int32)
            y = lax.bitcast_convert_type(jnp.int32(RSQRT_MAGIC) - (u >> 1), jnp.float32)
            y = y * (1.5 - 0.5 * v * y * y)
            y = y * (1.5 - 0.5 * v * y * y)
            y = y * (1.5 - 0.5 * v * y * y)
            disb[i, :] = jnp.where(v > 0.0, y, 0.0)
            return 0

        lax.fori_loop(0, RT, _rsqrt, 0)
        pltpu.sync_copy(disb, dis2.at[c, pl.ds(rbase, RT)])

        @pl.when(s < NS - 1)
        def _():
            pltpu.sync_copy(x3.at[pl.ds(rbase, RT), c], xbuf)
            _scale_rows_by_dis(xbuf, disb, 0, RT)
            pltpu.sync_copy(xbuf, y2.at[c, pl.ds(rbase, RT)])

        @pl.when(s == NS - 1)
        def _():
            base = (NS - 1) * RT
            pltpu.sync_copy(x3.at[pl.ds(base, NLAST), c], xbuf.at[pl.ds(0, NLAST)])
            _scale_rows_by_dis(xbuf, disb, 0, NLAST)
            pltpu.sync_copy(xbuf.at[pl.ds(0, NLAST)], y2.at[c, pl.ds(base, NLAST)])

    @functools.partial(
        pl.kernel,
        mesh=mesh,
        compiler_params=pltpu.CompilerParams(use_tc_tiling_on_sc=False),
        out_type=jax.ShapeDtypeStruct((N, NC, DH), jnp.float32),
        scratch_types=[
            pltpu.VMEM((BT, EBLK), jnp.int32),
            pltpu.VMEM((BT, EBLK), jnp.int32),
            pltpu.VMEM((EBLK, DH), jnp.float32),
            pltpu.VMEM((EBLK, DH), jnp.float32),
            pltpu.VMEM((WCH, DH), jnp.float32),
            pltpu.VMEM((WCH, 16), jnp.float32),
            pltpu.VMEM_SHARED((NPAD, DH), jnp.float32),
            pltpu.SemaphoreType.DMA,
            pltpu.SemaphoreType.DMA,
        ],
    )
    def aggregate(
        y2f, row2, col2, dis2, zeros128, out3,
        ridx, gidx, rows0, rows1, wbuf, disb, acc, sem0, sem1,
    ):
        c = lax.axis_index("c")
        s = lax.axis_index("s")
        rbase = s * RT
        ebase = s * BT

        pltpu.sync_copy(zeros128, acc.at[pl.ds(rbase, RT)])
        pltpu.sync_copy(row2.at[pl.ds(ebase, BT)], ridx)
        pltpu.sync_copy(col2.at[pl.ds(ebase, BT)], gidx)

        off = c * NPAD

        def _adj(j, _):
            for k in range(EBLK // 16):
                sl = pl.ds(16 * k, 16)
                gidx[j, sl] = gidx[j, sl] + off
            return 0

        lax.fori_loop(0, BT, _adj, 0)
        plsc.subcore_barrier()

        pltpu.make_async_copy(y2f.at[gidx.at[0]], rows0, sem0).start()

        def _step(jo, _):
            j = 2 * jo
            pltpu.make_async_copy(y2f.at[gidx.at[j]], rows0, sem0).wait()
            pltpu.make_async_copy(y2f.at[gidx.at[j + 1]], rows1, sem1).start()
            pltpu.sync_copy(rows0, acc.at[ridx.at[j]], add=True)
            pltpu.make_async_copy(y2f.at[gidx.at[j + 1]], rows1, sem1).wait()

            @pl.when(j + 2 < BT)
            def _():
                pltpu.make_async_copy(y2f.at[gidx.at[j + 2]], rows0, sem0).start()

            pltpu.sync_copy(rows1, acc.at[ridx.at[j + 1]], add=True)
            return 0

        lax.fori_loop(0, BT // 2, _step, 0)
        plsc.subcore_barrier()

        nrows = jnp.where(s < NS - 1, RT, NLAST)

        def _wb(ch, _):
            r0 = ch * WCH
            pltpu.sync_copy(acc.at[pl.ds(rbase + r0, WCH)], wbuf)
            pltpu.sync_copy(dis2.at[c, pl.ds(rbase + r0, WCH)], disb)
            _scale_rows_by_dis(wbuf, disb, 0, WCH)
            pltpu.sync_copy(wbuf, out3.at[pl.ds(rbase + r0, WCH), c])
            return 0

        lax.fori_loop(0, nrows // WCH, _wb, 0)

    return deg_prescale, aggregate


def kernel(x, edge_index):
    deg_prescale, aggregate = _build_kernels()
    ei = edge_index.astype(jnp.int32)
    npe = EPAD - E
    pad_row = N + (jnp.arange(npe, dtype=jnp.int32) % (NPAD - N))
    pad_col = jnp.arange(npe, dtype=jnp.int32) % N
    row2 = jnp.concatenate([ei[0], pad_row]).reshape(NBLK, EBLK)
    col2 = jnp.concatenate([ei[1], pad_col]).reshape(NBLK, EBLK)
    x3 = x.reshape(N, NC, DH)
    zeros16 = jnp.zeros((RT, 16), jnp.float32)
    zeros128 = jnp.zeros((RT, DH), jnp.float32)
    y2, dis2 = deg_prescale(x3, row2, zeros16)
    out3 = aggregate(y2.reshape(NC * NPAD, DH), row2, col2, dis2, zeros128)
    return out3.reshape(N, D)
